# baseline (speedup 1.0000x reference)
import functools

import jax
import jax.numpy as jnp
from jax import lax
from jax.experimental import pallas as pl
from jax.experimental.pallas import tpu as pltpu
from jax.experimental.pallas import tpu_sc as plsc

VOCAB = 100000
EMB = 64
MAX_NORM = 1.0
B = 1024
CTX = 50

NUM_CORES = 2
NUM_SUBCORES = 16
NUM_WORKERS = NUM_CORES * NUM_SUBCORES
BPW = B // NUM_WORKERS
LANES = 16
EV = EMB // LANES

_RSQRT_MAGIC = 0x5F3759DF


def _rsqrt_newton(v):
    y = lax.bitcast_convert_type(
        jnp.int32(_RSQRT_MAGIC) - (lax.bitcast_convert_type(v, jnp.int32) >> 1),
        jnp.float32)
    for _ in range(3):
        y = y * (1.5 - 0.5 * v * y * y)
    return y


def _lane_shuffle(v, idx):
    return lax.gather(
        v, idx[:, None],
        lax.GatherDimensionNumbers(offset_dims=(), collapsed_slice_dims=(0,),
                                   start_index_map=(0,)),
        slice_sizes=(1,), mode=lax.GatherScatterMode.PROMISE_IN_BOUNDS)


def _lane_allsum(v):
    lanes = lax.iota(jnp.int32, LANES)
    for sh in (8, 4, 2, 1):
        v = v + _lane_shuffle(v, lanes ^ sh)
    return v


def _pool_body(idx_hbm, table_hbm, x_hbm, idx_v, rows_a, rows_b, x_v,
               sem_a, sem_b):
    wid = lax.axis_index("s") * NUM_CORES + lax.axis_index("c")
    b0 = wid * BPW
    pltpu.sync_copy(idx_hbm.at[pl.ds(b0, BPW), :], idx_v)

    bufs = (rows_a, rows_b)
    sems = (sem_a, sem_b)
    copies = [None, None]
    copies[0] = pltpu.async_copy(table_hbm.at[idx_v.at[0]], bufs[0], sems[0])

    inv_ctx = jnp.float32(1.0 / CTX)

    for i in range(BPW):
        if i + 1 < BPW:
            j = (i + 1) % 2
            copies[j] = pltpu.async_copy(table_hbm.at[idx_v.at[i + 1]],
                                         bufs[j], sems[j])
        copies[i % 2].wait()
        rows = bufs[i % 2]

        def row_step(r, accs, rows=rows):
            e = [rows[r, pl.ds(k * LANES, LANES)] for k in range(EV)]
            ss = e[0] * e[0]
            for k in range(1, EV):
                ss = ss + e[k] * e[k]
            tot_v = _lane_allsum(ss)
            scale = jnp.minimum(jnp.float32(MAX_NORM),
                                MAX_NORM * _rsqrt_newton(tot_v))
            return tuple(a + ek * scale for a, ek in zip(accs, e))

        zero = jnp.zeros((LANES,), jnp.float32)
        accs = lax.fori_loop(0, CTX, row_step, (zero,) * EV)
        for k in range(EV):
            x_v[i, pl.ds(k * LANES, LANES)] = accs[k] * inv_ctx

    pltpu.sync_copy(x_v, x_hbm.at[pl.ds(b0, BPW), :])


@functools.partial(jax.jit, static_argnames=())
def _pool(inputs_, emb_table):
    mesh = plsc.VectorSubcoreMesh(core_axis_name="c", subcore_axis_name="s")
    return pl.kernel(
        _pool_body,
        out_type=jax.ShapeDtypeStruct((B, EMB), jnp.float32),
        mesh=mesh,
        scratch_types=[
            pltpu.VMEM((BPW, CTX), jnp.int32),
            pltpu.VMEM((CTX, EMB), jnp.float32),
            pltpu.VMEM((CTX, EMB), jnp.float32),
            pltpu.VMEM((BPW, EMB), jnp.float32),
            pltpu.SemaphoreType.DMA,
            pltpu.SemaphoreType.DMA,
        ],
        compiler_params=pltpu.CompilerParams(use_tc_tiling_on_sc=False),
    )(inputs_, emb_table)


V_TILE = 1024


def _mm_body(x_ref, w_ref, b_ref, o_ref):
    o_ref[...] = lax.dot_general(
        x_ref[...], w_ref[...],
        dimension_numbers=(((1,), (1,)), ((), ())),
        preferred_element_type=jnp.float32) + b_ref[...]


def _project(x, W, b2):
    grid = (pl.cdiv(VOCAB, V_TILE),)
    return pl.pallas_call(
        _mm_body,
        grid=grid,
        in_specs=[
            pl.BlockSpec((B, EMB), lambda i: (0, 0)),
            pl.BlockSpec((V_TILE, EMB), lambda i: (i, 0)),
            pl.BlockSpec((1, V_TILE), lambda i: (0, i)),
        ],
        out_specs=pl.BlockSpec((B, V_TILE), lambda i: (0, i)),
        out_shape=jax.ShapeDtypeStruct((B, VOCAB), jnp.float32),
    )(x, W, b2)


def kernel(inputs_, emb_table, W, b):
    x = _pool(inputs_, emb_table)
    return _project(x, W, b.reshape(1, VOCAB))


# speedup vs baseline: 2.2072x; 2.2072x over previous
import functools

import jax
import jax.numpy as jnp
from jax import lax
from jax.experimental import pallas as pl
from jax.experimental.pallas import tpu as pltpu
from jax.experimental.pallas import tpu_sc as plsc

VOCAB = 100000
EMB = 64
MAX_NORM = 1.0
B = 1024
CTX = 50

NUM_CORES = 2
NUM_SUBCORES = 16
NUM_WORKERS = NUM_CORES * NUM_SUBCORES
BPW = B // NUM_WORKERS
LANES = 16
EV = EMB // LANES

_RSQRT_MAGIC = 0x5F3759DF


def _rsqrt_newton(v):
    y = lax.bitcast_convert_type(
        jnp.int32(_RSQRT_MAGIC) - (lax.bitcast_convert_type(v, jnp.int32) >> 1),
        jnp.float32)
    for _ in range(3):
        y = y * (1.5 - 0.5 * v * y * y)
    return y


def _lane_shuffle(v, idx):
    return lax.gather(
        v, idx[:, None],
        lax.GatherDimensionNumbers(offset_dims=(), collapsed_slice_dims=(0,),
                                   start_index_map=(0,)),
        slice_sizes=(1,), mode=lax.GatherScatterMode.PROMISE_IN_BOUNDS)


def _lane_allsum(v):
    lanes = lax.iota(jnp.int32, LANES)
    for sh in (8, 4, 2, 1):
        v = v + _lane_shuffle(v, lanes ^ sh)
    return v


def _pool_body(idx_hbm, table_hbm, x_hbm, idx_v, rows_a, rows_b, x_v,
               sem_a, sem_b):
    wid = lax.axis_index("s") * NUM_CORES + lax.axis_index("c")
    b0 = wid * BPW
    pltpu.sync_copy(idx_hbm.at[pl.ds(b0, BPW), :], idx_v)

    bufs = (rows_a, rows_b)
    sems = (sem_a, sem_b)
    copies = [None, None]
    copies[0] = pltpu.async_copy(table_hbm.at[idx_v.at[0]], bufs[0], sems[0])

    inv_ctx = jnp.float32(1.0 / CTX)

    for i in range(BPW):
        if i + 1 < BPW:
            j = (i + 1) % 2
            copies[j] = pltpu.async_copy(table_hbm.at[idx_v.at[i + 1]],
                                         bufs[j], sems[j])
        copies[i % 2].wait()
        rows = bufs[i % 2]

        def row_step(r, accs, rows=rows):
            e = [rows[r, pl.ds(k * LANES, LANES)] for k in range(EV)]
            ss = e[0] * e[0]
            for k in range(1, EV):
                ss = ss + e[k] * e[k]
            tot_v = _lane_allsum(ss)
            scale = jnp.minimum(jnp.float32(MAX_NORM),
                                MAX_NORM * _rsqrt_newton(tot_v))
            return tuple(a + ek * scale for a, ek in zip(accs, e))

        zero = jnp.zeros((LANES,), jnp.float32)
        accs = lax.fori_loop(0, CTX, row_step, (zero,) * EV)
        for k in range(EV):
            x_v[i, pl.ds(k * LANES, LANES)] = accs[k] * inv_ctx

    pltpu.sync_copy(x_v, x_hbm.at[pl.ds(b0, BPW), :])


@functools.partial(jax.jit, static_argnames=())
def _pool(inputs_, emb_table):
    mesh = plsc.VectorSubcoreMesh(core_axis_name="c", subcore_axis_name="s")
    return pl.kernel(
        _pool_body,
        out_type=jax.ShapeDtypeStruct((B, EMB), jnp.float32),
        mesh=mesh,
        scratch_types=[
            pltpu.VMEM((BPW, CTX), jnp.int32),
            pltpu.VMEM((CTX, EMB), jnp.float32),
            pltpu.VMEM((CTX, EMB), jnp.float32),
            pltpu.VMEM((BPW, EMB), jnp.float32),
            pltpu.SemaphoreType.DMA,
            pltpu.SemaphoreType.DMA,
        ],
        compiler_params=pltpu.CompilerParams(use_tc_tiling_on_sc=False),
    )(inputs_, emb_table)


V_TILE = 1024


def _mm_body(x_ref, w_ref, b_ref, o_ref):
    acc = lax.dot_general(
        w_ref[...], x_ref[...],
        dimension_numbers=(((1,), (1,)), ((), ())),
        preferred_element_type=jnp.float32)
    o_ref[...] = acc + jnp.transpose(b_ref[...])


def _project(x, W, b2):
    grid = (pl.cdiv(VOCAB, V_TILE),)
    return pl.pallas_call(
        _mm_body,
        grid=grid,
        in_specs=[
            pl.BlockSpec((B, EMB), lambda i: (0, 0)),
            pl.BlockSpec((V_TILE, EMB), lambda i: (i, 0)),
            pl.BlockSpec((1, V_TILE), lambda i: (0, i)),
        ],
        out_specs=pl.BlockSpec((V_TILE, B), lambda i: (i, 0)),
        out_shape=jax.ShapeDtypeStruct((VOCAB, B), jnp.float32),
    )(x, W, b2)


def kernel(inputs_, emb_table, W, b):
    x = _pool(inputs_, emb_table)
    return _project(x, W, b.reshape(1, VOCAB)).T


# speedup vs baseline: 2.3049x; 1.0443x over previous
import functools

import jax
import jax.numpy as jnp
from jax import lax
from jax.experimental import pallas as pl
from jax.experimental.pallas import tpu as pltpu
from jax.experimental.pallas import tpu_sc as plsc

VOCAB = 100000
EMB = 64
MAX_NORM = 1.0
B = 1024
CTX = 50

NUM_CORES = 2
NUM_SUBCORES = 16
NUM_WORKERS = NUM_CORES * NUM_SUBCORES
BPW = B // NUM_WORKERS
LANES = 16
EV = EMB // LANES
PAD = 2 * EMB

VP = 2048


def _sp_body(wt_ref, o_ref):
    w = wt_ref[...]
    ss = lax.dot_general(jnp.ones((1, EMB), jnp.float32), w * w,
                         dimension_numbers=(((1,), (0,)), ((), ())),
                         preferred_element_type=jnp.float32)
    norm = jnp.sqrt(ss)
    scale = jnp.minimum(1.0, MAX_NORM / jnp.maximum(norm, 1e-7)) * (1.0 / CTX)
    scaled = w * scale
    o_ref[:, pl.ds(0, EMB)] = jnp.transpose(scaled)
    o_ref[:, pl.ds(EMB, EMB)] = jnp.zeros((VP, EMB), jnp.float32)


def _scalepack(wt):
    return pl.pallas_call(
        _sp_body,
        grid=(pl.cdiv(VOCAB, VP),),
        in_specs=[pl.BlockSpec((EMB, VP), lambda i: (0, i))],
        out_specs=pl.BlockSpec((VP, PAD), lambda i: (i, 0)),
        out_shape=jax.ShapeDtypeStruct((VOCAB, PAD), jnp.float32),
    )(wt)


def _pool_body(idx_hbm, ptab_hbm, x_hbm, idx_v, rows_a, rows_b, x_v,
               sem_a, sem_b):
    wid = lax.axis_index("s") * NUM_CORES + lax.axis_index("c")
    b0 = wid * BPW
    pltpu.sync_copy(idx_hbm.at[pl.ds(b0, BPW), :], idx_v)

    bufs = (rows_a, rows_b)
    sems = (sem_a, sem_b)
    copies = [None, None]
    copies[0] = pltpu.async_copy(ptab_hbm.at[idx_v.at[0]], bufs[0], sems[0])

    for i in range(BPW):
        if i + 1 < BPW:
            j = (i + 1) % 2
            copies[j] = pltpu.async_copy(ptab_hbm.at[idx_v.at[i + 1]],
                                         bufs[j], sems[j])
        copies[i % 2].wait()
        rows = bufs[i % 2]

        def row_step(r, accs, rows=rows):
            return tuple(a + rows[r, pl.ds(k * LANES, LANES)]
                         for k, a in enumerate(accs))

        zero = jnp.zeros((LANES,), jnp.float32)
        accs = lax.fori_loop(0, CTX, row_step, (zero,) * EV)
        for k in range(EV):
            x_v[i, pl.ds(k * LANES, LANES)] = accs[k]

    pltpu.sync_copy(x_v, x_hbm.at[pl.ds(b0, BPW), :])


def _pool(inputs_, ptab):
    mesh = plsc.VectorSubcoreMesh(core_axis_name="c", subcore_axis_name="s")
    return pl.kernel(
        _pool_body,
        out_type=jax.ShapeDtypeStruct((B, EMB), jnp.float32),
        mesh=mesh,
        scratch_types=[
            pltpu.VMEM((BPW, CTX), jnp.int32),
            pltpu.VMEM((CTX, PAD), jnp.float32),
            pltpu.VMEM((CTX, PAD), jnp.float32),
            pltpu.VMEM((BPW, EMB), jnp.float32),
            pltpu.SemaphoreType.DMA,
            pltpu.SemaphoreType.DMA,
        ],
    )(inputs_, ptab)


V_TILE = 1024


def _mm_body(x_ref, w_ref, b_ref, o_ref):
    acc = lax.dot_general(
        w_ref[...], x_ref[...],
        dimension_numbers=(((1,), (1,)), ((), ())),
        preferred_element_type=jnp.float32)
    o_ref[...] = acc + jnp.transpose(b_ref[...])


def _project(x, W, b2):
    grid = (pl.cdiv(VOCAB, V_TILE),)
    return pl.pallas_call(
        _mm_body,
        grid=grid,
        in_specs=[
            pl.BlockSpec((B, EMB), lambda i: (0, 0)),
            pl.BlockSpec((V_TILE, EMB), lambda i: (i, 0)),
            pl.BlockSpec((1, V_TILE), lambda i: (0, i)),
        ],
        out_specs=pl.BlockSpec((V_TILE, B), lambda i: (i, 0)),
        out_shape=jax.ShapeDtypeStruct((VOCAB, B), jnp.float32),
    )(x, W, b2)


def kernel(inputs_, emb_table, W, b):
    ptab = _scalepack(emb_table.T)
    x = _pool(inputs_, ptab)
    return _project(x, W, b.reshape(1, VOCAB)).T


# speedup vs baseline: 2.5149x; 1.0911x over previous
import functools

import jax
import jax.numpy as jnp
from jax import lax
from jax.experimental import pallas as pl
from jax.experimental.pallas import tpu as pltpu
from jax.experimental.pallas import tpu_sc as plsc

VOCAB = 100000
EMB = 64
MAX_NORM = 1.0
B = 1024
CTX = 50

NUM_CORES = 2
NUM_SUBCORES = 16
NUM_WORKERS = NUM_CORES * NUM_SUBCORES
BPW = B // NUM_WORKERS
LANES = 16
EV = EMB // LANES
PAD = 2 * EMB

VP = 2048


def _sp_body(wt_ref, o_ref):
    w = wt_ref[...]
    ss = lax.dot_general(jnp.ones((1, EMB), jnp.float32), w * w,
                         dimension_numbers=(((1,), (0,)), ((), ())),
                         preferred_element_type=jnp.float32)
    norm = jnp.sqrt(ss)
    scale = jnp.minimum(1.0, MAX_NORM / jnp.maximum(norm, 1e-7)) * (1.0 / CTX)
    scaled = w * scale
    o_ref[:, pl.ds(0, EMB)] = jnp.transpose(scaled)
    o_ref[:, pl.ds(EMB, EMB)] = jnp.zeros((VP, EMB), jnp.float32)


def _scalepack(wt):
    return pl.pallas_call(
        _sp_body,
        grid=(pl.cdiv(VOCAB, VP),),
        in_specs=[pl.BlockSpec((EMB, VP), lambda i: (0, i))],
        out_specs=pl.BlockSpec((VP, PAD), lambda i: (i, 0)),
        out_shape=jax.ShapeDtypeStruct((VOCAB, PAD), jnp.float32),
    )(wt)


def _pool_body(idx_hbm, ptab_hbm, x_hbm, idx_v, rows_a, rows_b, x_v,
               sem_a, sem_b):
    wid = lax.axis_index("s") * NUM_CORES + lax.axis_index("c")
    b0 = wid * BPW
    pltpu.sync_copy(idx_hbm.at[pl.ds(b0, BPW), :], idx_v)

    bufs = (rows_a, rows_b)
    sems = (sem_a, sem_b)
    copies = [None, None]
    copies[0] = pltpu.async_copy(ptab_hbm.at[idx_v.at[0]], bufs[0], sems[0])

    for i in range(BPW):
        if i + 1 < BPW:
            j = (i + 1) % 2
            copies[j] = pltpu.async_copy(ptab_hbm.at[idx_v.at[i + 1]],
                                         bufs[j], sems[j])
        copies[i % 2].wait()
        rows = bufs[i % 2]

        def row_step(r, accs, rows=rows):
            return tuple(a + rows[r, pl.ds(k * LANES, LANES)]
                         for k, a in enumerate(accs))

        zero = jnp.zeros((LANES,), jnp.float32)
        accs = lax.fori_loop(0, CTX, row_step, (zero,) * EV)
        for k in range(EV):
            x_v[i, pl.ds(k * LANES, LANES)] = accs[k]

    pltpu.sync_copy(x_v, x_hbm.at[pl.ds(b0, BPW), :])


def _pool(inputs_, ptab):
    mesh = plsc.VectorSubcoreMesh(core_axis_name="c", subcore_axis_name="s")
    return pl.kernel(
        _pool_body,
        out_type=jax.ShapeDtypeStruct((B, EMB), jnp.float32),
        mesh=mesh,
        scratch_types=[
            pltpu.VMEM((BPW, CTX), jnp.int32),
            pltpu.VMEM((CTX, PAD), jnp.float32),
            pltpu.VMEM((CTX, PAD), jnp.float32),
            pltpu.VMEM((BPW, EMB), jnp.float32),
            pltpu.SemaphoreType.DMA,
            pltpu.SemaphoreType.DMA,
        ],
    )(inputs_, ptab)


V_TILE = 1024


def _mm_body(x_ref, w_ref, b_ref, o_ref):
    acc = lax.dot_general(
        w_ref[...], x_ref[...],
        dimension_numbers=(((0,), (1,)), ((), ())),
        preferred_element_type=jnp.float32)
    o_ref[...] = acc + jnp.transpose(b_ref[...])


def _project(x, wt, b2):
    grid = (pl.cdiv(VOCAB, V_TILE),)
    return pl.pallas_call(
        _mm_body,
        grid=grid,
        in_specs=[
            pl.BlockSpec((B, EMB), lambda i: (0, 0)),
            pl.BlockSpec((EMB, V_TILE), lambda i: (0, i)),
            pl.BlockSpec((1, V_TILE), lambda i: (0, i)),
        ],
        out_specs=pl.BlockSpec((V_TILE, B), lambda i: (i, 0)),
        out_shape=jax.ShapeDtypeStruct((VOCAB, B), jnp.float32),
    )(x, wt, b2)


def kernel(inputs_, emb_table, W, b):
    ptab = _scalepack(emb_table.T)
    x = _pool(inputs_, ptab)
    return _project(x, W.T, b.reshape(1, VOCAB)).T


# speedup vs baseline: 2.7707x; 1.1017x over previous
import functools

import jax
import jax.numpy as jnp
from jax import lax
from jax.experimental import pallas as pl
from jax.experimental.pallas import tpu as pltpu
from jax.experimental.pallas import tpu_sc as plsc

VOCAB = 100000
EMB = 64
MAX_NORM = 1.0
B = 1024
CTX = 50

NUM_CORES = 2
NUM_SUBCORES = 16
NUM_WORKERS = NUM_CORES * NUM_SUBCORES
BPW = B // NUM_WORKERS
LANES = 16
EV = EMB // LANES
PAD = 2 * EMB

VP = 2048


def _sp_body(wt_ref, o_ref):
    w = wt_ref[...]
    ss = lax.dot_general(jnp.ones((1, EMB), jnp.float32), w * w,
                         dimension_numbers=(((1,), (0,)), ((), ())),
                         preferred_element_type=jnp.float32)
    norm = jnp.sqrt(ss)
    scale = jnp.minimum(1.0, MAX_NORM / jnp.maximum(norm, 1e-7)) * (1.0 / CTX)
    scaled = w * scale
    scaled_t = lax.dot_general(scaled, jnp.eye(EMB, dtype=jnp.float32),
                               dimension_numbers=(((0,), (0,)), ((), ())),
                               preferred_element_type=jnp.float32)
    o_ref[:, pl.ds(0, EMB)] = scaled_t
    o_ref[:, pl.ds(EMB, EMB)] = jnp.zeros((VP, EMB), jnp.float32)


def _scalepack(wt):
    return pl.pallas_call(
        _sp_body,
        grid=(pl.cdiv(VOCAB, VP),),
        in_specs=[pl.BlockSpec((EMB, VP), lambda i: (0, i))],
        out_specs=pl.BlockSpec((VP, PAD), lambda i: (i, 0)),
        out_shape=jax.ShapeDtypeStruct((VOCAB, PAD), jnp.float32),
    )(wt)


def _pool_body(idx_hbm, ptab_hbm, x_hbm, idx_v, rows_a, rows_b, x_v,
               sem_a, sem_b):
    wid = lax.axis_index("s") * NUM_CORES + lax.axis_index("c")
    b0 = wid * BPW
    pltpu.sync_copy(idx_hbm.at[pl.ds(b0, BPW), :], idx_v)

    bufs = (rows_a, rows_b)
    sems = (sem_a, sem_b)
    copies = [None, None]
    copies[0] = pltpu.async_copy(ptab_hbm.at[idx_v.at[0]], bufs[0], sems[0])

    for i in range(BPW):
        if i + 1 < BPW:
            j = (i + 1) % 2
            copies[j] = pltpu.async_copy(ptab_hbm.at[idx_v.at[i + 1]],
                                         bufs[j], sems[j])
        copies[i % 2].wait()
        rows = bufs[i % 2]

        def row_step(r, accs, rows=rows):
            return tuple(a + rows[r, pl.ds(k * LANES, LANES)]
                         for k, a in enumerate(accs))

        zero = jnp.zeros((LANES,), jnp.float32)
        accs = lax.fori_loop(0, CTX, row_step, (zero,) * EV)
        for k in range(EV):
            x_v[i, pl.ds(k * LANES, LANES)] = accs[k]

    pltpu.sync_copy(x_v, x_hbm.at[pl.ds(b0, BPW), :])


def _pool(inputs_, ptab):
    mesh = plsc.VectorSubcoreMesh(core_axis_name="c", subcore_axis_name="s")
    return pl.kernel(
        _pool_body,
        out_type=jax.ShapeDtypeStruct((B, EMB), jnp.float32),
        mesh=mesh,
        scratch_types=[
            pltpu.VMEM((BPW, CTX), jnp.int32),
            pltpu.VMEM((CTX, PAD), jnp.float32),
            pltpu.VMEM((CTX, PAD), jnp.float32),
            pltpu.VMEM((BPW, EMB), jnp.float32),
            pltpu.SemaphoreType.DMA,
            pltpu.SemaphoreType.DMA,
        ],
    )(inputs_, ptab)


V_TILE = 2048


def _mm_body(x_ref, w_ref, b_ref, o_ref):
    acc = lax.dot_general(
        w_ref[...], x_ref[...],
        dimension_numbers=(((0,), (1,)), ((), ())),
        preferred_element_type=jnp.float32)
    o_ref[...] = acc + jnp.transpose(b_ref[...])


def _project(x, wt, b2):
    grid = (pl.cdiv(VOCAB, V_TILE),)
    return pl.pallas_call(
        _mm_body,
        grid=grid,
        in_specs=[
            pl.BlockSpec((B, EMB), lambda i: (0, 0)),
            pl.BlockSpec((EMB, V_TILE), lambda i: (0, i)),
            pl.BlockSpec((1, V_TILE), lambda i: (0, i)),
        ],
        out_specs=pl.BlockSpec((V_TILE, B), lambda i: (i, 0)),
        out_shape=jax.ShapeDtypeStruct((VOCAB, B), jnp.float32),
    )(x, wt, b2)


def kernel(inputs_, emb_table, W, b):
    ptab = _scalepack(emb_table.T)
    x = _pool(inputs_, ptab)
    return _project(x, W.T, b.reshape(1, VOCAB)).T


# speedup vs baseline: 3.1276x; 1.1288x over previous
import functools

import jax
import jax.numpy as jnp
from jax import lax
from jax.experimental import pallas as pl
from jax.experimental.pallas import tpu as pltpu
from jax.experimental.pallas import tpu_sc as plsc

VOCAB = 100000
EMB = 64
MAX_NORM = 1.0
B = 1024
CTX = 50

NUM_CORES = 2
NUM_SUBCORES = 16
NUM_WORKERS = NUM_CORES * NUM_SUBCORES
BPW = B // NUM_WORKERS
LANES = 16
EV = EMB // LANES
PAD = 2 * EMB

VP = 8192


def _sp_body(wt_ref, o_ref):
    w = wt_ref[...]
    ss = lax.dot_general(jnp.ones((1, EMB), jnp.float32), w * w,
                         dimension_numbers=(((1,), (0,)), ((), ())),
                         preferred_element_type=jnp.float32)
    norm = jnp.sqrt(ss)
    scale = jnp.minimum(1.0, MAX_NORM / jnp.maximum(norm, 1e-7)) * (1.0 / CTX)
    scaled = w * scale
    scaled_t = lax.dot_general(scaled, jnp.eye(EMB, dtype=jnp.float32),
                               dimension_numbers=(((0,), (0,)), ((), ())),
                               preferred_element_type=jnp.float32)
    o_ref[:, pl.ds(0, EMB)] = scaled_t


def _scalepack(wt):
    return pl.pallas_call(
        _sp_body,
        grid=(pl.cdiv(VOCAB, VP),),
        in_specs=[pl.BlockSpec((EMB, VP), lambda i: (0, i))],
        out_specs=pl.BlockSpec((VP, PAD), lambda i: (i, 0)),
        out_shape=jax.ShapeDtypeStruct((VOCAB, PAD), jnp.float32),
    )(wt)


def _pool_body(idx_hbm, ptab_hbm, x_hbm, idx_v, rows_a, rows_b, x_v,
               sem_a, sem_b):
    wid = lax.axis_index("s") * NUM_CORES + lax.axis_index("c")
    b0 = wid * BPW
    pltpu.sync_copy(idx_hbm.at[pl.ds(b0, BPW), :], idx_v)

    bufs = (rows_a, rows_b)
    sems = (sem_a, sem_b)
    copies = [None, None]
    copies[0] = pltpu.async_copy(ptab_hbm.at[idx_v.at[0]], bufs[0], sems[0])

    for i in range(BPW):
        if i + 1 < BPW:
            j = (i + 1) % 2
            copies[j] = pltpu.async_copy(ptab_hbm.at[idx_v.at[i + 1]],
                                         bufs[j], sems[j])
        copies[i % 2].wait()
        rows = bufs[i % 2]

        def row_step(r, accs, rows=rows):
            return tuple(a + rows[r, pl.ds(k * LANES, LANES)]
                         for k, a in enumerate(accs))

        zero = jnp.zeros((LANES,), jnp.float32)
        accs = lax.fori_loop(0, CTX, row_step, (zero,) * EV)
        for k in range(EV):
            x_v[i, pl.ds(k * LANES, LANES)] = accs[k]

    pltpu.sync_copy(x_v, x_hbm.at[pl.ds(b0, BPW), :])


def _pool(inputs_, ptab):
    mesh = plsc.VectorSubcoreMesh(core_axis_name="c", subcore_axis_name="s")
    return pl.kernel(
        _pool_body,
        out_type=jax.ShapeDtypeStruct((B, EMB), jnp.float32),
        mesh=mesh,
        scratch_types=[
            pltpu.VMEM((BPW, CTX), jnp.int32),
            pltpu.VMEM((CTX, PAD), jnp.float32),
            pltpu.VMEM((CTX, PAD), jnp.float32),
            pltpu.VMEM((BPW, EMB), jnp.float32),
            pltpu.SemaphoreType.DMA,
            pltpu.SemaphoreType.DMA,
        ],
    )(inputs_, ptab)


V_TILE = 4096


def _mm_body(x_ref, w_ref, b_ref, o_ref):
    acc = lax.dot_general(
        w_ref[...], x_ref[...],
        dimension_numbers=(((0,), (1,)), ((), ())),
        preferred_element_type=jnp.float32)
    o_ref[...] = acc + jnp.transpose(b_ref[...])


def _project(x, wt, b2):
    grid = (pl.cdiv(VOCAB, V_TILE),)
    return pl.pallas_call(
        _mm_body,
        grid=grid,
        in_specs=[
            pl.BlockSpec((B, EMB), lambda i: (0, 0)),
            pl.BlockSpec((EMB, V_TILE), lambda i: (0, i)),
            pl.BlockSpec((1, V_TILE), lambda i: (0, i)),
        ],
        out_specs=pl.BlockSpec((V_TILE, B), lambda i: (i, 0)),
        out_shape=jax.ShapeDtypeStruct((VOCAB, B), jnp.float32),
    )(x, wt, b2)


def kernel(inputs_, emb_table, W, b):
    ptab = _scalepack(emb_table.T)
    x = _pool(inputs_, ptab)
    return _project(x, W.T, b.reshape(1, VOCAB)).T


# speedup vs baseline: 3.1892x; 1.0197x over previous
import functools

import jax
import jax.numpy as jnp
from jax import lax
from jax.experimental import pallas as pl
from jax.experimental.pallas import tpu as pltpu
from jax.experimental.pallas import tpu_sc as plsc

VOCAB = 100000
EMB = 64
MAX_NORM = 1.0
B = 1024
CTX = 50

NUM_CORES = 2
NUM_SUBCORES = 16
NUM_WORKERS = NUM_CORES * NUM_SUBCORES
BPW = B // NUM_WORKERS
LANES = 16
EV = EMB // LANES
PAD = 2 * EMB

VP = 8192


def _sp_body(wt_ref, o_ref):
    w = wt_ref[...]
    ss = lax.dot_general(jnp.ones((1, EMB), jnp.float32), w * w,
                         dimension_numbers=(((1,), (0,)), ((), ())),
                         preferred_element_type=jnp.float32)
    norm = jnp.sqrt(ss)
    scale = jnp.minimum(1.0, MAX_NORM / jnp.maximum(norm, 1e-7)) * (1.0 / CTX)
    scaled = w * scale
    scaled_t = lax.dot_general(scaled, jnp.eye(EMB, dtype=jnp.float32),
                               dimension_numbers=(((0,), (0,)), ((), ())),
                               preferred_element_type=jnp.float32)
    o_ref[:, pl.ds(0, EMB)] = scaled_t


def _scalepack(wt):
    return pl.pallas_call(
        _sp_body,
        grid=(pl.cdiv(VOCAB, VP),),
        in_specs=[pl.BlockSpec((EMB, VP), lambda i: (0, i))],
        out_specs=pl.BlockSpec((VP, PAD), lambda i: (i, 0)),
        out_shape=jax.ShapeDtypeStruct((VOCAB, PAD), jnp.float32),
    )(wt)


IPG = 2
NG = BPW // IPG


def _pool_body(idx_hbm, ptab_hbm, x_hbm, idx_v, rows_a, rows_b, x_v,
               sem_a, sem_b):
    wid = lax.axis_index("s") * NUM_CORES + lax.axis_index("c")
    b0 = wid * BPW
    g0 = wid * NG
    pltpu.sync_copy(idx_hbm.at[pl.ds(g0, NG), :], idx_v)

    bufs = (rows_a, rows_b)
    sems = (sem_a, sem_b)
    copies = [None, None]
    copies[0] = pltpu.async_copy(ptab_hbm.at[idx_v.at[0]], bufs[0], sems[0])

    for g in range(NG):
        if g + 1 < NG:
            j = (g + 1) % 2
            copies[j] = pltpu.async_copy(ptab_hbm.at[idx_v.at[g + 1]],
                                         bufs[j], sems[j])
        copies[g % 2].wait()
        rows = bufs[g % 2]

        for s in range(IPG):
            def row_step(r, accs, rows=rows, s=s):
                return tuple(a + rows[s * CTX + r, pl.ds(k * LANES, LANES)]
                             for k, a in enumerate(accs))

            zero = jnp.zeros((LANES,), jnp.float32)
            accs = lax.fori_loop(0, CTX, row_step, (zero,) * EV)
            for k in range(EV):
                x_v[g * IPG + s, pl.ds(k * LANES, LANES)] = accs[k]

    pltpu.sync_copy(x_v, x_hbm.at[pl.ds(b0, BPW), :])


def _pool(idx2d, ptab):
    mesh = plsc.VectorSubcoreMesh(core_axis_name="c", subcore_axis_name="s")
    return pl.kernel(
        _pool_body,
        out_type=jax.ShapeDtypeStruct((B, EMB), jnp.float32),
        mesh=mesh,
        scratch_types=[
            pltpu.VMEM((NG, IPG * CTX), jnp.int32),
            pltpu.VMEM((IPG * CTX, PAD), jnp.float32),
            pltpu.VMEM((IPG * CTX, PAD), jnp.float32),
            pltpu.VMEM((BPW, EMB), jnp.float32),
            pltpu.SemaphoreType.DMA,
            pltpu.SemaphoreType.DMA,
        ],
    )(idx2d, ptab)


V_TILE = 4096


def _mm_body(x_ref, w_ref, b_ref, o_ref):
    acc = lax.dot_general(
        w_ref[...], x_ref[...],
        dimension_numbers=(((0,), (1,)), ((), ())),
        preferred_element_type=jnp.float32)
    o_ref[...] = acc + jnp.transpose(b_ref[...])


def _project(x, wt, b2):
    grid = (pl.cdiv(VOCAB, V_TILE),)
    return pl.pallas_call(
        _mm_body,
        grid=grid,
        in_specs=[
            pl.BlockSpec((B, EMB), lambda i: (0, 0)),
            pl.BlockSpec((EMB, V_TILE), lambda i: (0, i)),
            pl.BlockSpec((1, V_TILE), lambda i: (0, i)),
        ],
        out_specs=pl.BlockSpec((V_TILE, B), lambda i: (i, 0)),
        out_shape=jax.ShapeDtypeStruct((VOCAB, B), jnp.float32),
    )(x, wt, b2)


def kernel(inputs_, emb_table, W, b):
    ptab = _scalepack(emb_table.T)
    x = _pool(inputs_.reshape(B // IPG, IPG * CTX), ptab)
    return _project(x, W.T, b.reshape(1, VOCAB)).T


# speedup vs baseline: 3.2655x; 1.0239x over previous
import functools

import jax
import jax.numpy as jnp
from jax import lax
from jax.experimental import pallas as pl
from jax.experimental.pallas import tpu as pltpu
from jax.experimental.pallas import tpu_sc as plsc

VOCAB = 100000
EMB = 64
MAX_NORM = 1.0
B = 1024
CTX = 50

NUM_CORES = 2
NUM_SUBCORES = 16
NUM_WORKERS = NUM_CORES * NUM_SUBCORES
BPW = B // NUM_WORKERS
LANES = 16
EV = EMB // LANES
PAD = 2 * EMB

VP = 8192


def _sp_body(wt_ref, o_ref):
    w = wt_ref[...]
    ss = lax.dot_general(jnp.ones((1, EMB), jnp.float32), w * w,
                         dimension_numbers=(((1,), (0,)), ((), ())),
                         preferred_element_type=jnp.float32)
    norm = jnp.sqrt(ss)
    scale = jnp.minimum(1.0, MAX_NORM / jnp.maximum(norm, 1e-7)) * (1.0 / CTX)
    scaled = w * scale
    scaled_t = lax.dot_general(scaled, jnp.eye(EMB, dtype=jnp.float32),
                               dimension_numbers=(((0,), (0,)), ((), ())),
                               preferred_element_type=jnp.float32)
    o_ref[:, pl.ds(0, EMB)] = scaled_t


def _scalepack(wt):
    return pl.pallas_call(
        _sp_body,
        grid=(pl.cdiv(VOCAB, VP),),
        in_specs=[pl.BlockSpec((EMB, VP), lambda i: (0, i))],
        out_specs=pl.BlockSpec((VP, PAD), lambda i: (i, 0)),
        out_shape=jax.ShapeDtypeStruct((VOCAB, PAD), jnp.float32),
    )(wt)


IPG = 2
NG = BPW // IPG


NBUF = 4


def _pool_body(idx_hbm, ptab_hbm, x_hbm, idx_v, rows_a, rows_b, rows_c,
               rows_d, x_v, sem_a, sem_b, sem_c, sem_d):
    wid = lax.axis_index("s") * NUM_CORES + lax.axis_index("c")
    b0 = wid * BPW
    g0 = wid * NG
    pltpu.sync_copy(idx_hbm.at[pl.ds(g0, NG), :], idx_v)

    bufs = (rows_a, rows_b, rows_c, rows_d)
    sems = (sem_a, sem_b, sem_c, sem_d)
    copies = [None] * NBUF

    def start(g):
        j = g % NBUF
        copies[j] = pltpu.async_copy(ptab_hbm.at[idx_v.at[g]], bufs[j],
                                     sems[j])

    for g in range(NBUF - 1):
        start(g)

    for g in range(NG):
        if g + NBUF - 1 < NG:
            start(g + NBUF - 1)
        copies[g % NBUF].wait()
        rows = bufs[g % NBUF]

        for s in range(IPG):
            def row_step(r, accs, rows=rows, s=s):
                return tuple(a + rows[s * CTX + r, pl.ds(k * LANES, LANES)]
                             for k, a in enumerate(accs))

            zero = jnp.zeros((LANES,), jnp.float32)
            accs = lax.fori_loop(0, CTX, row_step, (zero,) * EV)
            for k in range(EV):
                x_v[g * IPG + s, pl.ds(k * LANES, LANES)] = accs[k]

    pltpu.sync_copy(x_v, x_hbm.at[pl.ds(b0, BPW), :])


def _pool(idx2d, ptab):
    mesh = plsc.VectorSubcoreMesh(core_axis_name="c", subcore_axis_name="s")
    return pl.kernel(
        _pool_body,
        out_type=jax.ShapeDtypeStruct((B, EMB), jnp.float32),
        mesh=mesh,
        scratch_types=[
            pltpu.VMEM((NG, IPG * CTX), jnp.int32),
            pltpu.VMEM((IPG * CTX, PAD), jnp.float32),
            pltpu.VMEM((IPG * CTX, PAD), jnp.float32),
            pltpu.VMEM((IPG * CTX, PAD), jnp.float32),
            pltpu.VMEM((IPG * CTX, PAD), jnp.float32),
            pltpu.VMEM((BPW, EMB), jnp.float32),
            pltpu.SemaphoreType.DMA,
            pltpu.SemaphoreType.DMA,
            pltpu.SemaphoreType.DMA,
            pltpu.SemaphoreType.DMA,
        ],
    )(idx2d, ptab)


V_TILE = 4096


def _mm_body(x_ref, w_ref, b_ref, o_ref):
    acc = lax.dot_general(
        w_ref[...], x_ref[...],
        dimension_numbers=(((0,), (1,)), ((), ())),
        preferred_element_type=jnp.float32)
    o_ref[...] = acc + jnp.transpose(b_ref[...])


def _project(x, wt, b2):
    grid = (pl.cdiv(VOCAB, V_TILE),)
    return pl.pallas_call(
        _mm_body,
        grid=grid,
        in_specs=[
            pl.BlockSpec((B, EMB), lambda i: (0, 0)),
            pl.BlockSpec((EMB, V_TILE), lambda i: (0, i)),
            pl.BlockSpec((1, V_TILE), lambda i: (0, i)),
        ],
        out_specs=pl.BlockSpec((V_TILE, B), lambda i: (i, 0)),
        out_shape=jax.ShapeDtypeStruct((VOCAB, B), jnp.float32),
    )(x, wt, b2)


def kernel(inputs_, emb_table, W, b):
    ptab = _scalepack(emb_table.T)
    x = _pool(inputs_.reshape(B // IPG, IPG * CTX), ptab)
    return _project(x, W.T, b.reshape(1, VOCAB)).T
